# baseline (speedup 1.0000x reference)
import functools

import jax
import jax.numpy as jnp
from jax import lax
from jax.experimental import pallas as pl
from jax.experimental.pallas import tpu as pltpu
from jax.experimental.pallas import tpu_sc as plsc

N = 50000
DEG = 16
D = 100
V = 100000

NC = 2
NS = 16
NW = NC * NS
LANE = 128
SPW = 4
CHUNK = SPW * LANE

IT_N = 4
IT_E = 49
PAD_N = NW * IT_N * CHUNK
PAD_E = NW * IT_E * CHUNK

BLK = 512
NB = PAD_E // (BLK * DEG)
NP = NB * BLK


def _sc_gather_body(emb_hbm, idxn_hbm, idxe_hbm, outn_hbm, oute_hbm,
                    idx_v, rows_v, sem):
    wid = lax.axis_index("c") * NS + lax.axis_index("s")

    def run(idx_hbm, out_hbm, iters):
        def step(t, carry):
            pltpu.sync_copy(idx_hbm.at[wid, t], idx_v)
            copies = [
                pltpu.async_copy(emb_hbm.at[idx_v.at[j]],
                                 rows_v.at[pl.ds(j * LANE, LANE)], sem)
                for j in range(SPW)
            ]
            for cp in copies:
                cp.wait()
            row0 = (wid * iters + t) * CHUNK
            pltpu.sync_copy(rows_v, out_hbm.at[pl.ds(row0, CHUNK)])
            return carry

        lax.fori_loop(0, iters, step, 0)

    run(idxn_hbm, outn_hbm, IT_N)
    run(idxe_hbm, oute_hbm, IT_E)


@functools.cache
def _sc_gather():
    return pl.kernel(
        _sc_gather_body,
        out_type=[
            jax.ShapeDtypeStruct((PAD_N, D), jnp.float32),
            jax.ShapeDtypeStruct((PAD_E, D), jnp.float32),
        ],
        mesh=plsc.VectorSubcoreMesh(core_axis_name="c",
                                    subcore_axis_name="s",
                                    num_cores=NC, num_subcores=NS),
        scratch_types=[
            pltpu.VMEM((SPW, LANE), jnp.int32),
            pltpu.VMEM((CHUNK, D), jnp.float32),
            pltpu.SemaphoreType.DMA,
        ],
        compiler_params=pltpu.CompilerParams(use_tc_tiling_on_sc=False),
    )


def _tc_body(x0_ref, hne_ref, s_ref, wei_ref, w1a_ref, w2x_ref, w2m_ref,
             p_ref, o_ref):
    f32 = jnp.float32
    hi = lax.Precision.HIGHEST
    h3 = hne_ref[...]
    s = s_ref[...]
    wei = wei_ref[...]
    w1last = p_ref[0, :]
    b1 = p_ref[1, :]
    q1 = p_ref[2, :]
    b2 = p_ref[3, :]

    a2 = (h3 * s[:, None, :]).reshape(BLK * DEG, D)
    lin = jnp.dot(a2, w1a_ref[...], preferred_element_type=f32, precision=hi)
    pre = (lin.reshape(BLK, DEG, D)
           + wei[:, :, None] * w1last[None, None, :]
           + b1[None, None, :])
    h = jnp.tanh(pre)
    score = jnp.sum(h * q1[None, None, :], axis=2)
    score = score - jnp.max(score, axis=1, keepdims=True)
    e = jnp.exp(score)
    att = e / jnp.sum(e, axis=1, keepdims=True)
    msg = jnp.sum(att[:, :, None] * h3, axis=1)

    msgw = jnp.dot(msg, w2m_ref[...], preferred_element_type=f32,
                   precision=hi) + b2[None, :]
    x = x0_ref[...]
    x = jnp.maximum(
        jnp.dot(x, w2x_ref[...], preferred_element_type=f32, precision=hi)
        + msgw, 0.0)
    x = jnp.maximum(
        jnp.dot(x, w2x_ref[...], preferred_element_type=f32, precision=hi)
        + msgw, 0.0)
    o_ref[...] = x


def _tc_call(x0g, hne3, s_pad, wei_pad, w1a, w2x, w2m, p):
    return pl.pallas_call(
        _tc_body,
        grid=(NB,),
        in_specs=[
            pl.BlockSpec((BLK, D), lambda i: (i, 0)),
            pl.BlockSpec((BLK, DEG, D), lambda i: (i, 0, 0)),
            pl.BlockSpec((BLK, D), lambda i: (i, 0)),
            pl.BlockSpec((BLK, DEG), lambda i: (i, 0)),
            pl.BlockSpec((D, D), lambda i: (0, 0)),
            pl.BlockSpec((D, D), lambda i: (0, 0)),
            pl.BlockSpec((D, D), lambda i: (0, 0)),
            pl.BlockSpec((8, D), lambda i: (0, 0)),
        ],
        out_specs=pl.BlockSpec((BLK, D), lambda i: (i, 0)),
        out_shape=jax.ShapeDtypeStruct((NP, D), jnp.float32),
        compiler_params=pltpu.CompilerParams(
            dimension_semantics=("arbitrary",)),
    )(x0g, hne3, s_pad, wei_pad, w1a, w2x, w2m, p)


def kernel(nodes, nei, wei, s_vec, emb, W1_w, W1_b, q1_w, W2_w, W2_b):
    i32 = jnp.int32
    idxn = jnp.zeros((PAD_N,), i32).at[:N].set(nodes.astype(i32))
    idxe = jnp.zeros((PAD_E,), i32).at[:N * DEG].set(
        nei.reshape(-1).astype(i32))
    idxn = idxn.reshape(NW, IT_N, SPW, LANE)
    idxe = idxe.reshape(NW, IT_E, SPW, LANE)

    x0g, hneg = _sc_gather()(emb, idxn, idxe)
    hne3 = hneg.reshape(NP, DEG, D)

    s_pad = jnp.zeros((NP, D), jnp.float32).at[:N].set(s_vec)
    wei_pad = jnp.zeros((NP, DEG), jnp.float32).at[:N].set(wei)

    w1a = W1_w[:, :D].T
    w2x = W2_w[:, :D].T
    w2m = W2_w[:, D:].T
    p = jnp.zeros((8, D), jnp.float32)
    p = p.at[0].set(W1_w[:, D])
    p = p.at[1].set(W1_b)
    p = p.at[2].set(q1_w[0])
    p = p.at[3].set(W2_b)

    out = _tc_call(x0g, hne3, s_pad, wei_pad, w1a, w2x, w2m, p)
    return out[:N]


# speedup vs baseline: 1.1970x; 1.1970x over previous
import functools

import jax
import jax.numpy as jnp
from jax import lax
from jax.experimental import pallas as pl
from jax.experimental.pallas import tpu as pltpu
from jax.experimental.pallas import tpu_sc as plsc

N = 50000
DEG = 16
D = 100
DP = 128
V = 100000

NC = 2
NS = 16
NW = NC * NS
LANE = 128
SPW = 2
CHUNK = SPW * LANE

IT_N = 8
IT_E = 98
PAD_N = NW * IT_N * CHUNK
PAD_E = NW * IT_E * CHUNK

BLK = 512
NB = PAD_E // (BLK * DEG)
NP = NB * BLK


def _sc_gather_body(emb_hbm, idxn_hbm, idxe_hbm, outn_hbm, oute_hbm,
                    idxn_v, idxe_v, rows0, rows1, gsem0, gsem1,
                    osem0, osem1):
    wid = lax.axis_index("c") * NS + lax.axis_index("s")
    pltpu.sync_copy(idxn_hbm.at[wid], idxn_v)
    pltpu.sync_copy(idxe_hbm.at[wid], idxe_v)
    rows = (rows0, rows1)
    gsem = (gsem0, gsem1)
    osem = (osem0, osem1)

    def run(idx_v, out_hbm, iters):
        def gather(t, b):
            for j in range(SPW):
                pltpu.async_copy(emb_hbm.at[idx_v.at[t * SPW + j]],
                                 rows[b].at[pl.ds(j * LANE, LANE)],
                                 gsem[b])

        def gather_wait(t, b):
            for j in range(SPW):
                pltpu.make_async_copy(emb_hbm.at[idx_v.at[t * SPW + j]],
                                      rows[b].at[pl.ds(j * LANE, LANE)],
                                      gsem[b]).wait()

        def store(t, b):
            row0 = (wid * iters + t) * CHUNK
            return pltpu.async_copy(rows[b],
                                    out_hbm.at[pl.ds(row0, CHUNK)],
                                    osem[b])

        gather(0, 0)
        gather(1, 1)

        def step(i, carry):
            for b in range(2):
                t = 2 * i + b
                gather_wait(t, b)
                cp = store(t, b)

                @pl.when(t + 2 < iters)
                def _():
                    cp.wait()
                    gather(t + 2, b)

                @pl.when(t + 2 >= iters)
                def _():
                    cp.wait()
            return carry

        lax.fori_loop(0, iters // 2, step, 0)

    run(idxn_v, outn_hbm, IT_N)
    run(idxe_v, oute_hbm, IT_E)


@functools.cache
def _sc_gather():
    return pl.kernel(
        _sc_gather_body,
        out_type=[
            jax.ShapeDtypeStruct((PAD_N, DP), jnp.float32),
            jax.ShapeDtypeStruct((PAD_E, DP), jnp.float32),
        ],
        mesh=plsc.VectorSubcoreMesh(core_axis_name="c",
                                    subcore_axis_name="s",
                                    num_cores=NC, num_subcores=NS),
        scratch_types=[
            pltpu.VMEM((IT_N * SPW, LANE), jnp.int32),
            pltpu.VMEM((IT_E * SPW, LANE), jnp.int32),
            pltpu.VMEM((CHUNK, DP), jnp.float32),
            pltpu.VMEM((CHUNK, DP), jnp.float32),
            pltpu.SemaphoreType.DMA,
            pltpu.SemaphoreType.DMA,
            pltpu.SemaphoreType.DMA,
            pltpu.SemaphoreType.DMA,
        ],
    )


def _tc_body(x0_ref, hne_ref, s_ref, wei_ref, w1a_ref, w2x_ref, w2m_ref,
             p_ref, o_ref):
    f32 = jnp.float32
    hi = lax.Precision.HIGHEST
    h3 = hne_ref[...]
    s = s_ref[...]
    wei = wei_ref[...]
    w1last = p_ref[0, :]
    b1 = p_ref[1, :]
    q1 = p_ref[2, :]
    b2 = p_ref[3, :]

    a2 = (h3 * s[:, None, :]).reshape(BLK * DEG, DP)
    lin = jnp.dot(a2, w1a_ref[...], preferred_element_type=f32, precision=hi)
    pre = (lin.reshape(BLK, DEG, DP)
           + wei[:, :, None] * w1last[None, None, :]
           + b1[None, None, :])
    h = jnp.tanh(pre)
    score = jnp.sum(h * q1[None, None, :], axis=2)
    score = score - jnp.max(score, axis=1, keepdims=True)
    e = jnp.exp(score)
    att = e / jnp.sum(e, axis=1, keepdims=True)
    msg = jnp.sum(att[:, :, None] * h3, axis=1)

    msgw = jnp.dot(msg, w2m_ref[...], preferred_element_type=f32,
                   precision=hi) + b2[None, :]
    x = x0_ref[...]
    x = jnp.maximum(
        jnp.dot(x, w2x_ref[...], preferred_element_type=f32, precision=hi)
        + msgw, 0.0)
    x = jnp.maximum(
        jnp.dot(x, w2x_ref[...], preferred_element_type=f32, precision=hi)
        + msgw, 0.0)
    o_ref[...] = x[:, :D]


def _tc_call(x0g, hne3, s_pad, wei_pad, w1a, w2x, w2m, p):
    return pl.pallas_call(
        _tc_body,
        grid=(NB,),
        in_specs=[
            pl.BlockSpec((BLK, DP), lambda i: (i, 0)),
            pl.BlockSpec((BLK, DEG, DP), lambda i: (i, 0, 0)),
            pl.BlockSpec((BLK, DP), lambda i: (i, 0)),
            pl.BlockSpec((BLK, DEG), lambda i: (i, 0)),
            pl.BlockSpec((DP, DP), lambda i: (0, 0)),
            pl.BlockSpec((DP, DP), lambda i: (0, 0)),
            pl.BlockSpec((DP, DP), lambda i: (0, 0)),
            pl.BlockSpec((8, DP), lambda i: (0, 0)),
        ],
        out_specs=pl.BlockSpec((BLK, D), lambda i: (i, 0)),
        out_shape=jax.ShapeDtypeStruct((NP, D), jnp.float32),
        compiler_params=pltpu.CompilerParams(
            dimension_semantics=("arbitrary",)),
    )(x0g, hne3, s_pad, wei_pad, w1a, w2x, w2m, p)


def kernel(nodes, nei, wei, s_vec, emb, W1_w, W1_b, q1_w, W2_w, W2_b):
    i32 = jnp.int32
    f32 = jnp.float32
    idxn = jnp.zeros((PAD_N,), i32).at[:N].set(nodes.astype(i32))
    idxe = jnp.zeros((PAD_E,), i32).at[:N * DEG].set(
        nei.reshape(-1).astype(i32))
    idxn = idxn.reshape(NW, IT_N * SPW, LANE)
    idxe = idxe.reshape(NW, IT_E * SPW, LANE)
    embp = jnp.pad(emb, ((0, 0), (0, DP - D)))

    x0g, hneg = _sc_gather()(embp, idxn, idxe)
    hne3 = hneg.reshape(NP, DEG, DP)

    s_pad = jnp.zeros((NP, DP), f32).at[:N, :D].set(s_vec)
    wei_pad = jnp.zeros((NP, DEG), f32).at[:N].set(wei)

    w1a = jnp.zeros((DP, DP), f32).at[:D, :D].set(W1_w[:, :D].T)
    w2x = jnp.zeros((DP, DP), f32).at[:D, :D].set(W2_w[:, :D].T)
    w2m = jnp.zeros((DP, DP), f32).at[:D, :D].set(W2_w[:, D:].T)
    p = jnp.zeros((8, DP), f32)
    p = p.at[0, :D].set(W1_w[:, D])
    p = p.at[1, :D].set(W1_b)
    p = p.at[2, :D].set(q1_w[0])
    p = p.at[3, :D].set(W2_b)

    out = _tc_call(x0g, hne3, s_pad, wei_pad, w1a, w2x, w2m, p)
    return out[:N]


# speedup vs baseline: 3.0390x; 2.5387x over previous
import functools

import jax
import jax.numpy as jnp
from jax import lax
from jax.experimental import pallas as pl
from jax.experimental.pallas import tpu as pltpu
from jax.experimental.pallas import tpu_sc as plsc

N = 50000
DEG = 16
D = 100
DP = 128
V = 100000

NC = 2
NS = 16
NW = NC * NS
LANE = 128
SPW = 2
CHUNK = SPW * LANE
NBUF = 3

IT_N = 9
IT_E = 99
PAD_N = NW * IT_N * CHUNK
PAD_E = NW * IT_E * CHUNK

BLK = 512
NB = PAD_E // (BLK * DEG)
NP = NB * BLK


def _sc_gather_body(emb_hbm, idxn_hbm, idxe_hbm, outn_hbm, oute_hbm,
                    idxn_v, idxe_v, rows0, rows1, rows2,
                    gsem0, gsem1, gsem2, osem0, osem1, osem2):
    wid = lax.axis_index("c") * NS + lax.axis_index("s")
    pltpu.sync_copy(idxn_hbm.at[wid], idxn_v)
    pltpu.sync_copy(idxe_hbm.at[wid], idxe_v)
    rows = (rows0, rows1, rows2)
    gsem = (gsem0, gsem1, gsem2)
    osem = (osem0, osem1, osem2)

    def run(idx_v, out_hbm, iters):
        def gather(t, b):
            for j in range(SPW):
                pltpu.async_copy(emb_hbm.at[idx_v.at[t * SPW + j]],
                                 rows[b].at[pl.ds(j * LANE, LANE)],
                                 gsem[b])

        def gather_wait(t, b):
            for j in range(SPW):
                pltpu.make_async_copy(emb_hbm.at[idx_v.at[t * SPW + j]],
                                      rows[b].at[pl.ds(j * LANE, LANE)],
                                      gsem[b]).wait()

        def store(t, b):
            row0 = (t * NW + wid) * CHUNK
            pltpu.async_copy(rows[b], out_hbm.at[pl.ds(row0, CHUNK)],
                             osem[b])

        def store_wait(t, b):
            row0 = (t * NW + wid) * CHUNK
            pltpu.make_async_copy(rows[b], out_hbm.at[pl.ds(row0, CHUNK)],
                                  osem[b]).wait()

        gather(0, 0)
        gather(1, 1)

        def step(i, carry):
            for b in range(NBUF):
                t = i * NBUF + b
                gather_wait(t, b)
                store(t, b)
                nb = (b + 2) % NBUF

                @pl.when(t == 0)
                def _():
                    gather(2, 2)

                @pl.when((t >= 1) & (t + 2 < iters))
                def _():
                    store_wait(t - 1, nb)
                    gather(t + 2, nb)
            return carry

        lax.fori_loop(0, iters // NBUF, step, 0)
        store_wait(iters - 3, 0)
        store_wait(iters - 2, 1)
        store_wait(iters - 1, 2)

    run(idxn_v, outn_hbm, IT_N)
    run(idxe_v, oute_hbm, IT_E)


@functools.cache
def _sc_gather():
    return pl.kernel(
        _sc_gather_body,
        out_type=[
            jax.ShapeDtypeStruct((PAD_N, DP), jnp.float32),
            jax.ShapeDtypeStruct((PAD_E, DP), jnp.float32),
        ],
        mesh=plsc.VectorSubcoreMesh(core_axis_name="c",
                                    subcore_axis_name="s",
                                    num_cores=NC, num_subcores=NS),
        scratch_types=[
            pltpu.VMEM((IT_N * SPW, LANE), jnp.int32),
            pltpu.VMEM((IT_E * SPW, LANE), jnp.int32),
            pltpu.VMEM((CHUNK, DP), jnp.float32),
            pltpu.VMEM((CHUNK, DP), jnp.float32),
            pltpu.VMEM((CHUNK, DP), jnp.float32),
            pltpu.SemaphoreType.DMA,
            pltpu.SemaphoreType.DMA,
            pltpu.SemaphoreType.DMA,
            pltpu.SemaphoreType.DMA,
            pltpu.SemaphoreType.DMA,
            pltpu.SemaphoreType.DMA,
        ],
        compiler_params=pltpu.CompilerParams(use_tc_tiling_on_sc=True),
    )


def _tc_body(x0_ref, hne_ref, s_ref, wei_ref, w1a_ref, w2x_ref, w2m_ref,
             p_ref, o_ref):
    f32 = jnp.float32
    h3 = hne_ref[...]
    s = s_ref[...]
    wei = wei_ref[...]
    w1last = p_ref[0, :]
    b1 = p_ref[1, :]
    q1 = p_ref[2, :]
    b2 = p_ref[3, :]

    a2 = (h3 * s[:, None, :]).reshape(BLK * DEG, DP)
    lin = jnp.dot(a2, w1a_ref[...], preferred_element_type=f32)
    pre = (lin.reshape(BLK, DEG, DP)
           + wei[:, :, None] * w1last[None, None, :]
           + b1[None, None, :])
    h = jnp.tanh(pre)
    score = jnp.sum(h * q1[None, None, :], axis=2)
    e = jnp.exp(score)
    att = e / jnp.sum(e, axis=1, keepdims=True)
    msg = jnp.sum(att[:, :, None] * h3, axis=1)

    msgw = jnp.dot(msg, w2m_ref[...], preferred_element_type=f32) + b2[None, :]
    x = x0_ref[...]
    x = jnp.maximum(
        jnp.dot(x, w2x_ref[...], preferred_element_type=f32) + msgw, 0.0)
    x = jnp.maximum(
        jnp.dot(x, w2x_ref[...], preferred_element_type=f32) + msgw, 0.0)
    o_ref[...] = x[:, :D]


def _tc_call(x0g, hne3, s_pad, wei_pad, w1a, w2x, w2m, p):
    return pl.pallas_call(
        _tc_body,
        grid=(NB,),
        in_specs=[
            pl.BlockSpec((BLK, DP), lambda i: (i, 0)),
            pl.BlockSpec((BLK, DEG, DP), lambda i: (i, 0, 0)),
            pl.BlockSpec((BLK, DP), lambda i: (i, 0)),
            pl.BlockSpec((BLK, DEG), lambda i: (i, 0)),
            pl.BlockSpec((DP, DP), lambda i: (0, 0)),
            pl.BlockSpec((DP, DP), lambda i: (0, 0)),
            pl.BlockSpec((DP, DP), lambda i: (0, 0)),
            pl.BlockSpec((8, DP), lambda i: (0, 0)),
        ],
        out_specs=pl.BlockSpec((BLK, D), lambda i: (i, 0)),
        out_shape=jax.ShapeDtypeStruct((NP, D), jnp.float32),
        compiler_params=pltpu.CompilerParams(
            dimension_semantics=("arbitrary",)),
    )(x0g, hne3, s_pad, wei_pad, w1a, w2x, w2m, p)


def _worker_major(idx_flat, iters):
    return (idx_flat.reshape(iters, NW, CHUNK).transpose(1, 0, 2)
            .reshape(NW, iters * SPW, LANE))


def kernel(nodes, nei, wei, s_vec, emb, W1_w, W1_b, q1_w, W2_w, W2_b):
    i32 = jnp.int32
    f32 = jnp.float32
    idxn = (jnp.arange(PAD_N, dtype=i32) % V).at[:N].set(nodes.astype(i32))
    idxe = (jnp.arange(PAD_E, dtype=i32) % V).at[:N * DEG].set(
        nei.reshape(-1).astype(i32))
    idxn = _worker_major(idxn, IT_N)
    idxe = _worker_major(idxe, IT_E)
    embp = jnp.pad(emb, ((0, 0), (0, DP - D)))

    x0g, hneg = _sc_gather()(embp, idxn, idxe)
    hne3 = hneg.reshape(NP, DEG, DP)

    s_pad = jnp.zeros((NP, DP), f32).at[:N, :D].set(s_vec)
    wei_pad = jnp.zeros((NP, DEG), f32).at[:N].set(wei)

    w1a = jnp.zeros((DP, DP), f32).at[:D, :D].set(W1_w[:, :D].T)
    w2x = jnp.zeros((DP, DP), f32).at[:D, :D].set(W2_w[:, :D].T)
    w2m = jnp.zeros((DP, DP), f32).at[:D, :D].set(W2_w[:, D:].T)
    p = jnp.zeros((8, DP), f32)
    p = p.at[0, :D].set(W1_w[:, D])
    p = p.at[1, :D].set(W1_b)
    p = p.at[2, :D].set(q1_w[0])
    p = p.at[3, :D].set(W2_b)

    out = _tc_call(x0g, hne3, s_pad, wei_pad, w1a, w2x, w2m, p)
    return out[:N]
